# Initial kernel scaffold; baseline (speedup 1.0000x reference)
import jax
import jax.numpy as jnp
from jax.experimental import pallas as pl


def kernel(x, pos, W, b, batch):
    raise NotImplementedError("write your pallas kernel here")


# speedup vs baseline: 1.8952x; 1.8952x over previous
import functools

import jax
import jax.numpy as jnp
from jax import lax
from jax.experimental import pallas as pl
from jax.experimental.pallas import tpu as pltpu
from jax.experimental.pallas import tpu_sc as plsc

NSEG = 16
DF = 64
DX = 61
DP = 3
LANES = 16

NC = 2
NS = 16
NW = NC * NS

BLK = 4096
CHUNK = 512


def _mlp_body(x_ref, pos_ref, wx_ref, wp_ref, b_ref, h_ref):
    h = lax.dot_general(x_ref[...], wx_ref[...],
                        (((1,), (0,)), ((), ())),
                        preferred_element_type=jnp.float32)
    h = h + lax.dot_general(pos_ref[...], wp_ref[...],
                            (((1,), (0,)), ((), ())),
                            preferred_element_type=jnp.float32)
    h_ref[...] = jnp.maximum(h + b_ref[...], 0.0)


def _mlp(x, pos, wx, wp, b2):
    n = x.shape[0]
    grid = n // BLK
    return pl.pallas_call(
        _mlp_body,
        grid=(grid,),
        in_specs=[
            pl.BlockSpec((BLK, DX), lambda i: (i, 0)),
            pl.BlockSpec((BLK, DP), lambda i: (i, 0)),
            pl.BlockSpec((DX, DF), lambda i: (0, 0)),
            pl.BlockSpec((DP, DF), lambda i: (0, 0)),
            pl.BlockSpec((1, DF), lambda i: (0, 0)),
        ],
        out_specs=pl.BlockSpec((BLK, DF), lambda i: (i, 0)),
        out_shape=jax.ShapeDtypeStruct((n, DF), jnp.float32),
    )(x, pos, wx, wp, b2)


def _segmax_body(n, h_hbm, starts_hbm, ends_hbm, out_hbm,
                 buf, acc, sidx, eidx):
    rpw = n // NW
    nchunks = rpw // CHUNK
    wid = lax.axis_index("s") * NC + lax.axis_index("c")
    base = wid * rpw

    pltpu.sync_copy(starts_hbm, sidx)
    pltpu.sync_copy(ends_hbm, eidx)

    neg_inf = jnp.full((LANES,), -jnp.inf, jnp.float32)
    for s in range(NSEG):
        for j in range(DF // LANES):
            acc[s, pl.ds(j * LANES, LANES)] = neg_inf

    sv = sidx[...]
    ev = eidx[...]
    lo = []
    hi = []
    for s in range(NSEG):
        lo.append(jnp.clip(sv[s] - base, 0, rpw))
        hi.append(jnp.clip(ev[s] - base, 0, rpw))

    for k in range(nchunks):
        pltpu.sync_copy(h_hbm.at[pl.ds(base + k * CHUNK, CHUNK)], buf)
        for s in range(NSEG):
            a = jnp.clip(lo[s] - k * CHUNK, 0, CHUNK)
            z = jnp.clip(hi[s] - k * CHUNK, 0, CHUNK)

            @pl.when(z > a)
            def _(s=s, a=a, z=z):
                carry = tuple(acc[s, pl.ds(j * LANES, LANES)]
                              for j in range(DF // LANES))

                def body(r, cur):
                    return tuple(
                        jnp.maximum(cur[j], buf[r, pl.ds(j * LANES, LANES)])
                        for j in range(DF // LANES))

                res = lax.fori_loop(a, z, body, carry)
                for j in range(DF // LANES):
                    acc[s, pl.ds(j * LANES, LANES)] = res[j]

    pltpu.sync_copy(acc, out_hbm.at[wid])


def _segmax(h, starts, ends):
    n = h.shape[0]
    mesh = plsc.VectorSubcoreMesh(core_axis_name="c", subcore_axis_name="s",
                                  num_cores=NC, num_subcores=NS)
    f = pl.kernel(
        functools.partial(_segmax_body, n),
        out_type=jax.ShapeDtypeStruct((NW, NSEG, DF), jnp.float32),
        mesh=mesh,
        scratch_types=[
            pltpu.VMEM((CHUNK, DF), jnp.float32),
            pltpu.VMEM((NSEG, DF), jnp.float32),
            pltpu.VMEM((LANES,), jnp.int32),
            pltpu.VMEM((LANES,), jnp.int32),
        ],
    )
    return f(h, starts, ends)


def _combine_body(p_ref, o_ref):
    o_ref[...] = jnp.max(p_ref[...], axis=0)


def _combine(partials):
    return pl.pallas_call(
        _combine_body,
        out_shape=jax.ShapeDtypeStruct((NSEG, DF), jnp.float32),
    )(partials)


def kernel(x, pos, W, b, batch):
    n = x.shape[0]
    wx = W[:, :DX].T
    wp = W[:, DX:].T
    b2 = b.reshape(1, DF)
    seg = jnp.arange(NSEG, dtype=batch.dtype)
    starts = jnp.searchsorted(batch, seg, side="left").astype(jnp.int32)
    ends = jnp.concatenate(
        [starts[1:], jnp.array([n], jnp.int32)])

    h = _mlp(x, pos, wx, wp, b2)
    partials = _segmax(h, starts, ends)
    return _combine(partials)


# speedup vs baseline: 3.0285x; 1.5980x over previous
import functools

import jax
import jax.numpy as jnp
from jax import lax
from jax.experimental import pallas as pl
from jax.experimental.pallas import tpu as pltpu
from jax.experimental.pallas import tpu_sc as plsc

NSEG = 16
DF = 64
DF2 = 128
DX = 61
DP = 3
LANES = 16

NC = 2
NS = 16
NW = NC * NS

BLKC = 4096
CHUNK = 256


def _mlp_body(xa_ref, xb_ref, pa_ref, pb_ref, wx_ref, wp_ref, b_ref, h_ref):
    def half(xt_ref, pt_ref):
        h = lax.dot_general(xt_ref[...], wx_ref[...],
                            (((0,), (0,)), ((), ())),
                            preferred_element_type=jnp.float32)
        h = h + lax.dot_general(pt_ref[...], wp_ref[...],
                                (((0,), (0,)), ((), ())),
                                preferred_element_type=jnp.float32)
        return jnp.maximum(h + b_ref[...], 0.0)

    h_ref[...] = jnp.concatenate([half(xa_ref, pa_ref), half(xb_ref, pb_ref)],
                                 axis=1)


def _mlp(xt, post, wx, wp, b2):
    n = xt.shape[1]
    nh = n // 2
    grid = nh // BLKC
    noff = nh // BLKC
    return pl.pallas_call(
        _mlp_body,
        grid=(grid,),
        in_specs=[
            pl.BlockSpec((DX, BLKC), lambda i: (0, i)),
            pl.BlockSpec((DX, BLKC), lambda i, o=noff: (0, i + o)),
            pl.BlockSpec((DP, BLKC), lambda i: (0, i)),
            pl.BlockSpec((DP, BLKC), lambda i, o=noff: (0, i + o)),
            pl.BlockSpec((DX, DF), lambda i: (0, 0)),
            pl.BlockSpec((DP, DF), lambda i: (0, 0)),
            pl.BlockSpec((1, DF), lambda i: (0, 0)),
        ],
        out_specs=pl.BlockSpec((BLKC, DF2), lambda i: (i, 0)),
        out_shape=jax.ShapeDtypeStruct((nh, DF2), jnp.float32),
    )(xt, xt, post, post, wx, wp, b2)


def _segmax_body(n, h_hbm, starts_hbm, ends_hbm, out_hbm,
                 buf, acc, sidx, eidx):
    nh = n // 2
    rpw = nh // NW
    nchunks = rpw // CHUNK
    wid = lax.axis_index("s") * NC + lax.axis_index("c")
    base = wid * rpw

    pltpu.sync_copy(starts_hbm, sidx)
    pltpu.sync_copy(ends_hbm, eidx)

    neg_inf = jnp.full((LANES,), -jnp.inf, jnp.float32)
    for s in range(NSEG):
        for j in range(DF // LANES):
            acc[s, pl.ds(j * LANES, LANES)] = neg_inf

    sv = sidx[...]
    ev = eidx[...]
    lo0, hi0, lo1, hi1 = [], [], [], []
    for s in range(NSEG):
        lo0.append(jnp.clip(sv[s] - base, 0, rpw))
        hi0.append(jnp.clip(ev[s] - base, 0, rpw))
        lo1.append(jnp.clip(sv[s] - nh - base, 0, rpw))
        hi1.append(jnp.clip(ev[s] - nh - base, 0, rpw))

    nj = DF // LANES
    for k in range(nchunks):
        pltpu.sync_copy(h_hbm.at[pl.ds(base + k * CHUNK, CHUNK)], buf)
        for s in range(NSEG):
            for (lo, hi, coff) in ((lo0, hi0, 0), (lo1, hi1, DF)):
                a = jnp.clip(lo[s] - k * CHUNK, 0, CHUNK)
                z = jnp.clip(hi[s] - k * CHUNK, 0, CHUNK)

                @pl.when(z > a)
                def _(s=s, a=a, z=z, coff=coff):
                    carry = tuple(acc[s, pl.ds(j * LANES, LANES)]
                                  for j in range(nj))

                    def body(r, cur):
                        return tuple(
                            jnp.maximum(
                                cur[j],
                                buf[r, pl.ds(coff + j * LANES, LANES)])
                            for j in range(nj))

                    res = lax.fori_loop(a, z, body, carry)
                    for j in range(nj):
                        acc[s, pl.ds(j * LANES, LANES)] = res[j]

    pltpu.sync_copy(acc, out_hbm.at[wid])


def _segmax(h2, starts, ends):
    n = h2.shape[0] * 2
    mesh = plsc.VectorSubcoreMesh(core_axis_name="c", subcore_axis_name="s",
                                  num_cores=NC, num_subcores=NS)
    f = pl.kernel(
        functools.partial(_segmax_body, n),
        out_type=jax.ShapeDtypeStruct((NW, NSEG, DF), jnp.float32),
        mesh=mesh,
        scratch_types=[
            pltpu.VMEM((CHUNK, DF2), jnp.float32),
            pltpu.VMEM((NSEG, DF), jnp.float32),
            pltpu.VMEM((LANES,), jnp.int32),
            pltpu.VMEM((LANES,), jnp.int32),
        ],
    )
    return f(h2, starts, ends)


def _combine_body(p_ref, o_ref):
    o_ref[...] = jnp.max(p_ref[...], axis=0)


def _combine(partials):
    return pl.pallas_call(
        _combine_body,
        out_shape=jax.ShapeDtypeStruct((NSEG, DF), jnp.float32),
    )(partials)


def kernel(x, pos, W, b, batch):
    n = x.shape[0]
    xt = x.T
    post = pos.T
    wx = W[:, :DX].T
    wp = W[:, DX:].T
    b2 = b.reshape(1, DF)
    seg = jnp.arange(NSEG, dtype=batch.dtype)
    starts = jnp.searchsorted(batch, seg, side="left").astype(jnp.int32)
    ends = jnp.concatenate(
        [starts[1:], jnp.array([n], jnp.int32)])

    h2 = _mlp(xt, post, wx, wp, b2)
    partials = _segmax(h2, starts, ends)
    return _combine(partials)


# speedup vs baseline: 3.8692x; 1.2776x over previous
import functools

import jax
import jax.numpy as jnp
from jax import lax
from jax.experimental import pallas as pl
from jax.experimental.pallas import tpu as pltpu
from jax.experimental.pallas import tpu_sc as plsc

NSEG = 16
DF = 64
DF2 = 128
DX = 61
DP = 3
LANES = 16

NC = 2
NS = 16
NW = NC * NS

BLKC = 4096
CHUNK = 256


def _mlp_body(xa_ref, xb_ref, pa_ref, pb_ref, wx_ref, wp_ref, b_ref, h_ref):
    def half(xt_ref, pt_ref):
        h = lax.dot_general(xt_ref[...], wx_ref[...],
                            (((0,), (0,)), ((), ())),
                            preferred_element_type=jnp.float32)
        h = h + lax.dot_general(pt_ref[...], wp_ref[...],
                                (((0,), (0,)), ((), ())),
                                preferred_element_type=jnp.float32)
        return jnp.maximum(h + b_ref[...], 0.0)

    h_ref[...] = jnp.concatenate([half(xa_ref, pa_ref), half(xb_ref, pb_ref)],
                                 axis=1)


def _mlp(xt, post, wx, wp, b2):
    n = xt.shape[1]
    nh = n // 2
    grid = nh // BLKC
    noff = nh // BLKC
    return pl.pallas_call(
        _mlp_body,
        grid=(grid,),
        in_specs=[
            pl.BlockSpec((DX, BLKC), lambda i: (0, i)),
            pl.BlockSpec((DX, BLKC), lambda i, o=noff: (0, i + o)),
            pl.BlockSpec((DP, BLKC), lambda i: (0, i)),
            pl.BlockSpec((DP, BLKC), lambda i, o=noff: (0, i + o)),
            pl.BlockSpec((DX, DF), lambda i: (0, 0)),
            pl.BlockSpec((DP, DF), lambda i: (0, 0)),
            pl.BlockSpec((1, DF), lambda i: (0, 0)),
        ],
        out_specs=pl.BlockSpec((BLKC, DF2), lambda i: (i, 0)),
        out_shape=jax.ShapeDtypeStruct((nh, DF2), jnp.float32),
    )(xt, xt, post, post, wx, wp, b2)


def _segmax_body(n, h_hbm, starts_hbm, ends_hbm, out_hbm,
                 buf0, buf1, acc, sidx, eidx, sem0, sem1):
    nh = n // 2
    rpw = nh // NW
    nchunks = rpw // CHUNK
    wid = lax.axis_index("s") * NC + lax.axis_index("c")
    base = wid * rpw

    pltpu.sync_copy(starts_hbm, sidx)
    pltpu.sync_copy(ends_hbm, eidx)

    neg_inf = jnp.full((LANES,), -jnp.inf, jnp.float32)
    for s in range(NSEG):
        for j in range(DF // LANES):
            acc[s, pl.ds(j * LANES, LANES)] = neg_inf

    sv = sidx[...]
    ev = eidx[...]
    lo0, hi0, lo1, hi1 = [], [], [], []
    for s in range(NSEG):
        lo0.append(jnp.clip(sv[s] - base, 0, rpw))
        hi0.append(jnp.clip(ev[s] - base, 0, rpw))
        lo1.append(jnp.clip(sv[s] - nh - base, 0, rpw))
        hi1.append(jnp.clip(ev[s] - nh - base, 0, rpw))

    lb = LANES
    nj = DF // lb
    bufs = (buf0, buf1)
    sems = (sem0, sem1)

    def start(k):
        return pltpu.async_copy(
            h_hbm.at[pl.ds(base + k * CHUNK, CHUNK)], bufs[k % 2],
            sems[k % 2])

    start(0)
    for k in range(nchunks):
        cp = start(k + 1) if k + 1 < nchunks else None
        pltpu.make_async_copy(
            h_hbm.at[pl.ds(base + k * CHUNK, CHUNK)], bufs[k % 2],
            sems[k % 2]).wait()
        buf = bufs[k % 2]
        for s in range(NSEG):
            for (lo, hi, coff) in ((lo0, hi0, 0), (lo1, hi1, DF)):
                a = jnp.clip(lo[s] - k * CHUNK, 0, CHUNK)
                z = jnp.clip(hi[s] - k * CHUNK, 0, CHUNK)

                @pl.when(z > a)
                def _(s=s, a=a, z=z, coff=coff, buf=buf):
                    carry = tuple(acc[s, pl.ds(j * lb, lb)]
                                  for j in range(nj))

                    def body(r, cur):
                        return tuple(
                            jnp.maximum(
                                cur[j],
                                buf[r, pl.ds(coff + j * lb, lb)])
                            for j in range(nj))

                    res = lax.fori_loop(a, z, body, carry)
                    for j in range(nj):
                        acc[s, pl.ds(j * lb, lb)] = res[j]

    pltpu.sync_copy(acc, out_hbm.at[wid])


def _segmax(h2, starts, ends):
    n = h2.shape[0] * 2
    mesh = plsc.VectorSubcoreMesh(core_axis_name="c", subcore_axis_name="s",
                                  num_cores=NC, num_subcores=NS)
    f = pl.kernel(
        functools.partial(_segmax_body, n),
        out_type=jax.ShapeDtypeStruct((NW, NSEG, DF), jnp.float32),
        mesh=mesh,
        scratch_types=[
            pltpu.VMEM((CHUNK, DF2), jnp.float32),
            pltpu.VMEM((CHUNK, DF2), jnp.float32),
            pltpu.VMEM((NSEG, DF), jnp.float32),
            pltpu.VMEM((LANES,), jnp.int32),
            pltpu.VMEM((LANES,), jnp.int32),
            pltpu.SemaphoreType.DMA,
            pltpu.SemaphoreType.DMA,
        ],
    )
    return f(h2, starts, ends)


def _combine_body(p_ref, o_ref):
    o_ref[...] = jnp.max(p_ref[...], axis=0)


def _combine(partials):
    return pl.pallas_call(
        _combine_body,
        out_shape=jax.ShapeDtypeStruct((NSEG, DF), jnp.float32),
    )(partials)


def kernel(x, pos, W, b, batch):
    n = x.shape[0]
    xt = x.T
    post = pos.T
    wx = W[:, :DX].T
    wp = W[:, DX:].T
    b2 = b.reshape(1, DF)
    seg = jnp.arange(NSEG, dtype=batch.dtype)
    starts = jnp.sum(batch[:, None] < seg[None, :], axis=0,
                     dtype=jnp.int32)
    ends = jnp.concatenate(
        [starts[1:], jnp.array([n], jnp.int32)])

    h2 = _mlp(xt, post, wx, wp, b2)
    partials = _segmax(h2, starts, ends)
    return _combine(partials)


# speedup vs baseline: 4.0433x; 1.0450x over previous
import functools

import jax
import jax.numpy as jnp
from jax import lax
from jax.experimental import pallas as pl
from jax.experimental.pallas import tpu as pltpu
from jax.experimental.pallas import tpu_sc as plsc

NSEG = 16
DF = 64
DF2 = 128
DX = 61
DP = 3
LANES = 16

NC = 2
NS = 16
NW = NC * NS

BLKC = 4096
CHUNK = 256


def _mlp_body(xa_ref, xb_ref, pa_ref, pb_ref, wx_ref, wp_ref, b_ref, h_ref):
    def half(xt_ref, pt_ref):
        h = lax.dot_general(xt_ref[...], wx_ref[...],
                            (((0,), (0,)), ((), ())),
                            preferred_element_type=jnp.float32)
        h = h + lax.dot_general(pt_ref[...], wp_ref[...],
                                (((0,), (0,)), ((), ())),
                                preferred_element_type=jnp.float32)
        return jnp.maximum(h + b_ref[...], 0.0)

    h_ref[...] = jnp.concatenate([half(xa_ref, pa_ref), half(xb_ref, pb_ref)],
                                 axis=1)


def _mlp(xt, post, wx, wp, b2, boff, rows):
    n = xt.shape[1]
    nh = n // 2
    grid = rows // BLKC
    noff = nh // BLKC
    return pl.pallas_call(
        _mlp_body,
        grid=(grid,),
        in_specs=[
            pl.BlockSpec((DX, BLKC), lambda i, o=boff: (0, i + o)),
            pl.BlockSpec((DX, BLKC), lambda i, o=boff + noff: (0, i + o)),
            pl.BlockSpec((DP, BLKC), lambda i, o=boff: (0, i + o)),
            pl.BlockSpec((DP, BLKC), lambda i, o=boff + noff: (0, i + o)),
            pl.BlockSpec((DX, DF), lambda i: (0, 0)),
            pl.BlockSpec((DP, DF), lambda i: (0, 0)),
            pl.BlockSpec((1, DF), lambda i: (0, 0)),
        ],
        out_specs=pl.BlockSpec((BLKC, DF2), lambda i: (i, 0)),
        out_shape=jax.ShapeDtypeStruct((rows, DF2), jnp.float32),
    )(xt, xt, post, post, wx, wp, b2)


def _segmax_body(n, row_off, h_hbm, starts_hbm, ends_hbm, out_hbm,
                 buf0, buf1, acc, sidx, eidx, sem0, sem1):
    nh = n // 2
    rows = h_hbm.shape[0]
    rpw = rows // NW
    nchunks = rpw // CHUNK
    wid = lax.axis_index("s") * NC + lax.axis_index("c")
    base = wid * rpw
    gbase = row_off + base

    pltpu.sync_copy(starts_hbm, sidx)
    pltpu.sync_copy(ends_hbm, eidx)

    neg_inf = jnp.full((LANES,), -jnp.inf, jnp.float32)
    for s in range(NSEG):
        for j in range(DF // LANES):
            acc[s, pl.ds(j * LANES, LANES)] = neg_inf

    sv = sidx[...]
    ev = eidx[...]
    lo0, hi0, lo1, hi1 = [], [], [], []
    for s in range(NSEG):
        lo0.append(jnp.clip(sv[s] - gbase, 0, rpw))
        hi0.append(jnp.clip(ev[s] - gbase, 0, rpw))
        lo1.append(jnp.clip(sv[s] - nh - gbase, 0, rpw))
        hi1.append(jnp.clip(ev[s] - nh - gbase, 0, rpw))

    nj = DF // LANES
    bufs = (buf0, buf1)
    sems = (sem0, sem1)

    def start(k):
        pltpu.async_copy(
            h_hbm.at[pl.ds(base + k * CHUNK, CHUNK)], bufs[k % 2],
            sems[k % 2])

    start(0)
    for k in range(nchunks):
        if k + 1 < nchunks:
            start(k + 1)
        pltpu.make_async_copy(
            h_hbm.at[pl.ds(base + k * CHUNK, CHUNK)], bufs[k % 2],
            sems[k % 2]).wait()
        buf = bufs[k % 2]
        for s in range(NSEG):
            for (lo, hi, coff) in ((lo0, hi0, 0), (lo1, hi1, DF)):
                a = jnp.clip(lo[s] - k * CHUNK, 0, CHUNK)
                z = jnp.clip(hi[s] - k * CHUNK, 0, CHUNK)

                @pl.when(z > a)
                def _(s=s, a=a, z=z, coff=coff, buf=buf):
                    carry = tuple(acc[s, pl.ds(j * LANES, LANES)]
                                  for j in range(nj))

                    def body(r, cur):
                        return tuple(
                            jnp.maximum(
                                cur[j],
                                buf[r, pl.ds(coff + j * LANES, LANES)])
                            for j in range(nj))

                    res = lax.fori_loop(a, z, body, carry)
                    for j in range(nj):
                        acc[s, pl.ds(j * LANES, LANES)] = res[j]

    pltpu.sync_copy(acc, out_hbm.at[wid])


def _segmax(h2, starts, ends, row_off, n):
    mesh = plsc.VectorSubcoreMesh(core_axis_name="c", subcore_axis_name="s",
                                  num_cores=NC, num_subcores=NS)
    f = pl.kernel(
        functools.partial(_segmax_body, n, row_off),
        out_type=jax.ShapeDtypeStruct((NW, NSEG, DF), jnp.float32),
        mesh=mesh,
        scratch_types=[
            pltpu.VMEM((CHUNK, DF2), jnp.float32),
            pltpu.VMEM((CHUNK, DF2), jnp.float32),
            pltpu.VMEM((NSEG, DF), jnp.float32),
            pltpu.VMEM((LANES,), jnp.int32),
            pltpu.VMEM((LANES,), jnp.int32),
            pltpu.SemaphoreType.DMA,
            pltpu.SemaphoreType.DMA,
        ],
    )
    return f(h2, starts, ends)


def _combine_body(pa_ref, pb_ref, o_ref):
    o_ref[...] = jnp.maximum(jnp.max(pa_ref[...], axis=0),
                             jnp.max(pb_ref[...], axis=0))


def _combine(pa, pb):
    return pl.pallas_call(
        _combine_body,
        out_shape=jax.ShapeDtypeStruct((NSEG, DF), jnp.float32),
    )(pa, pb)


def kernel(x, pos, W, b, batch):
    n = x.shape[0]
    nh = n // 2
    xt = x.T
    post = pos.T
    wx = W[:, :DX].T
    wp = W[:, DX:].T
    b2 = b.reshape(1, DF)
    seg = jnp.arange(NSEG, dtype=batch.dtype)
    starts = jnp.sum(batch[:, None] < seg[None, :], axis=0,
                     dtype=jnp.int32)
    ends = jnp.concatenate(
        [starts[1:], jnp.array([n], jnp.int32)])

    nblk = nh // BLKC
    h2a = _mlp(xt, post, wx, wp, b2, 0, nh // 2)
    pa = _segmax(h2a, starts, ends, 0, n)
    h2b = _mlp(xt, post, wx, wp, b2, nblk // 2, nh // 2)
    pb = _segmax(h2b, starts, ends, nh // 2, n)
    return _combine(pa, pb)
